# Initial kernel scaffold; baseline (speedup 1.0000x reference)
import jax
import jax.numpy as jnp
from jax.experimental import pallas as pl


def kernel(stu_id, exer_id, theta_w, a_w, b_w):
    raise NotImplementedError("write your pallas kernel here")


# speedup vs baseline: 1.0942x; 1.0942x over previous
import functools

import jax
import jax.numpy as jnp
from jax import lax
from jax.experimental import pallas as pl
from jax.experimental.pallas import tpu as pltpu
from jax.experimental.pallas import tpu_sc as plsc

_B = 16384
_L = 16
_NC = 2
_NS = 16
_NW = _NC * _NS
_BPW = _B // _NW
_CHUNK = 128
_NCH = _BPW // _CHUNK


def _irt_body(stu_hbm, exer_hbm, theta_hbm, a_hbm, b_hbm, out_hbm,
              sidx, eidx, th, av, bv, ov, sem):
    wid = lax.axis_index("s") * _NC + lax.axis_index("c")
    base = wid * _BPW
    for j in range(_NCH):
        pltpu.sync_copy(stu_hbm.at[pl.ds(base + j * _CHUNK, _CHUNK)], sidx.at[j])
        pltpu.sync_copy(exer_hbm.at[pl.ds(base + j * _CHUNK, _CHUNK)], eidx.at[j])
    copies = []
    for j in range(_NCH):
        copies.append(pltpu.async_copy(theta_hbm.at[sidx.at[j]], th.at[j], sem))
        copies.append(pltpu.async_copy(a_hbm.at[eidx.at[j]], av.at[j], sem))
        copies.append(pltpu.async_copy(b_hbm.at[eidx.at[j]], bv.at[j], sem))
    for c in copies:
        c.wait()
    for j in range(_NCH):
        for i in range(_CHUNK // _L):
            sl = pl.ds(i * _L, _L)
            t = th[j, sl]
            a_raw = av[j, sl]
            b_val = bv[j, sl]
            a2 = 2.0 / (1.0 + jnp.exp(-a_raw))
            z = 1.7 * a2 * (t - b_val)
            ov[pl.ds(j * _CHUNK + i * _L, _L)] = 1.0 / (1.0 + jnp.exp(-z))
    pltpu.sync_copy(ov, out_hbm.at[pl.ds(base, _BPW)])


@jax.jit
def kernel(stu_id, exer_id, theta_w, a_w, b_w):
    mesh = plsc.VectorSubcoreMesh(core_axis_name="c", subcore_axis_name="s")
    run = functools.partial(
        pl.kernel,
        mesh=mesh,
        out_type=jax.ShapeDtypeStruct((_B,), jnp.float32),
        scratch_types=[
            pltpu.VMEM((_NCH, _CHUNK), jnp.int32),
            pltpu.VMEM((_NCH, _CHUNK), jnp.int32),
            pltpu.VMEM((_NCH, _CHUNK), jnp.float32),
            pltpu.VMEM((_NCH, _CHUNK), jnp.float32),
            pltpu.VMEM((_NCH, _CHUNK), jnp.float32),
            pltpu.VMEM((_BPW,), jnp.float32),
            pltpu.SemaphoreType.DMA,
        ],
    )(_irt_body)
    return run(stu_id.astype(jnp.int32), exer_id.astype(jnp.int32),
               theta_w.reshape(-1), a_w.reshape(-1), b_w.reshape(-1))


# speedup vs baseline: 1.1356x; 1.0379x over previous
import functools

import jax
import jax.numpy as jnp
from jax import lax
from jax.experimental import pallas as pl
from jax.experimental.pallas import tpu as pltpu
from jax.experimental.pallas import tpu_sc as plsc

_B = 16384
_L = 16
_NC = 2
_NS = 16
_NW = _NC * _NS
_BPW = _B // _NW
_CHUNK = 128
_NCH = _BPW // _CHUNK


def _irt_body(stu_hbm, exer_hbm, theta_hbm, a_hbm, b_hbm, out_hbm,
              sidx, eidx, th, av, bv, ov, sem):
    wid = lax.axis_index("s") * _NC + lax.axis_index("c")
    base = wid * _BPW
    pltpu.sync_copy(stu_hbm.at[pl.ds(base, _BPW)], sidx)
    pltpu.sync_copy(exer_hbm.at[pl.ds(base, _BPW)], eidx)
    copies = []
    for j in range(_NCH):
        csl = pl.ds(j * _CHUNK, _CHUNK)
        copies.append(pltpu.async_copy(theta_hbm.at[sidx.at[csl]], th.at[j], sem))
        copies.append(pltpu.async_copy(a_hbm.at[eidx.at[csl]], av.at[j], sem))
        copies.append(pltpu.async_copy(b_hbm.at[eidx.at[csl]], bv.at[j], sem))
    for j in range(_NCH):
        copies[3 * j].wait()
        copies[3 * j + 1].wait()
        copies[3 * j + 2].wait()
        for i in range(_CHUNK // _L):
            sl = pl.ds(i * _L, _L)
            t = th[j, sl]
            a_raw = av[j, sl]
            b_val = bv[j, sl]
            a2 = 2.0 / (1.0 + jnp.exp(-a_raw))
            z = 1.7 * a2 * (t - b_val)
            ov[pl.ds(j * _CHUNK + i * _L, _L)] = 1.0 / (1.0 + jnp.exp(-z))
    pltpu.sync_copy(ov, out_hbm.at[pl.ds(base, _BPW)])


@jax.jit
def kernel(stu_id, exer_id, theta_w, a_w, b_w):
    mesh = plsc.VectorSubcoreMesh(core_axis_name="c", subcore_axis_name="s")
    run = functools.partial(
        pl.kernel,
        mesh=mesh,
        out_type=jax.ShapeDtypeStruct((_B,), jnp.float32),
        scratch_types=[
            pltpu.VMEM((_BPW,), jnp.int32),
            pltpu.VMEM((_BPW,), jnp.int32),
            pltpu.VMEM((_NCH, _CHUNK), jnp.float32),
            pltpu.VMEM((_NCH, _CHUNK), jnp.float32),
            pltpu.VMEM((_NCH, _CHUNK), jnp.float32),
            pltpu.VMEM((_BPW,), jnp.float32),
            pltpu.SemaphoreType.DMA,
        ],
    )(_irt_body)
    return run(stu_id.astype(jnp.int32), exer_id.astype(jnp.int32),
               theta_w.reshape(-1), a_w.reshape(-1), b_w.reshape(-1))


# speedup vs baseline: 1.1472x; 1.0102x over previous
import functools

import jax
import jax.numpy as jnp
from jax import lax
from jax.experimental import pallas as pl
from jax.experimental.pallas import tpu as pltpu
from jax.experimental.pallas import tpu_sc as plsc

_B = 16384
_L = 16
_NC = 2
_NS = 16
_NW = _NC * _NS
_BPW = _B // _NW
_CHUNK = 128
_NCH = _BPW // _CHUNK


def _irt_body(stu_hbm, exer_hbm, theta_hbm, a_hbm, b_hbm, out_hbm,
              sidx, eidx, th, av, bv, ov, sem_g, sem_idx, sem_out):
    wid = lax.axis_index("s") * _NC + lax.axis_index("c")
    stu_stage = pltpu.async_copy(stu_hbm.at[wid], sidx, sem_idx)
    exer_stage = pltpu.async_copy(exer_hbm.at[wid], eidx, sem_idx)
    stu_stage.wait()
    gathers = []
    for j in range(_NCH):
        gathers.append(pltpu.async_copy(theta_hbm.at[sidx.at[j]], th.at[j], sem_g))
    exer_stage.wait()
    for j in range(_NCH):
        gathers.append(pltpu.async_copy(a_hbm.at[eidx.at[j]], av.at[j], sem_g))
        gathers.append(pltpu.async_copy(b_hbm.at[eidx.at[j]], bv.at[j], sem_g))
    for g in gathers:
        g.wait()
    for j in range(_NCH):
        for i in range(_CHUNK // _L):
            sl = pl.ds(i * _L, _L)
            t = th[j, sl]
            a_raw = av[j, sl]
            b_val = bv[j, sl]
            z = 3.4 * (t - b_val) / (1.0 + jnp.exp(-a_raw))
            ov[j, sl] = 1.0 / (1.0 + jnp.exp(-z))
    pltpu.async_copy(ov, out_hbm.at[wid], sem_out).wait()


@jax.jit
def kernel(stu_id, exer_id, theta_w, a_w, b_w):
    mesh = plsc.VectorSubcoreMesh(core_axis_name="c", subcore_axis_name="s")
    run = functools.partial(
        pl.kernel,
        mesh=mesh,
        out_type=jax.ShapeDtypeStruct((_NW, _NCH, _CHUNK), jnp.float32),
        scratch_types=[
            pltpu.VMEM((_NCH, _CHUNK), jnp.int32),
            pltpu.VMEM((_NCH, _CHUNK), jnp.int32),
            pltpu.VMEM((_NCH, _CHUNK), jnp.float32),
            pltpu.VMEM((_NCH, _CHUNK), jnp.float32),
            pltpu.VMEM((_NCH, _CHUNK), jnp.float32),
            pltpu.VMEM((_NCH, _CHUNK), jnp.float32),
            pltpu.SemaphoreType.DMA,
            pltpu.SemaphoreType.DMA,
            pltpu.SemaphoreType.DMA,
        ],
    )(_irt_body)
    out = run(stu_id.astype(jnp.int32).reshape(_NW, _NCH, _CHUNK),
              exer_id.astype(jnp.int32).reshape(_NW, _NCH, _CHUNK),
              theta_w.reshape(-1), a_w.reshape(-1), b_w.reshape(-1))
    return out.reshape(-1)
